# baseline (speedup 1.0000x reference)
import jax
import jax.numpy as jnp
from jax import lax
from jax.experimental import pallas as pl
from jax.experimental.pallas import tpu as pltpu
from jax.experimental.pallas import tpu_sc as plsc

H = 512
N_T = 10000
N_E = 10000
E = 160000
NUM_GRAPHS = 64
NUM_CLASSES = 32

NW = 32
EPW_P = 5120
K = 64
NCH = EPW_P // K
PAD_E = NW * EPW_P - E
N_PAD = 10240
RPT = N_PAD // 16
DUMP = 10000

R_BLK = 1000
N_BLK = N_T // R_BLK


def _proj_body(xt, xe, wpt, wpe, ht, he):
  dnt = (((1,), (1,)), ((), ()))
  ht[...] = lax.dot_general(xt[...], wpt[...], dnt,
                            preferred_element_type=jnp.float32)
  he[...] = lax.dot_general(xe[...], wpe[...], dnt,
                            preferred_element_type=jnp.float32)


def _tc_project(x_trace, x_event, wpt, wpe):
  f32 = jnp.float32
  full = lambda shp: pl.BlockSpec(shp, lambda i: tuple(0 for _ in shp))
  return pl.pallas_call(
      _proj_body,
      grid=(N_BLK,),
      in_specs=[
          pl.BlockSpec((R_BLK, 256), lambda i: (i, 0)),
          pl.BlockSpec((R_BLK, 128), lambda i: (i, 0)),
          full((H, 256)), full((H, 128)),
      ],
      out_specs=[
          pl.BlockSpec((R_BLK, H), lambda i: (i, 0)),
          pl.BlockSpec((R_BLK, H), lambda i: (i, 0)),
      ],
      out_shape=(jax.ShapeDtypeStruct((N_T, H), f32),
                 jax.ShapeDtypeStruct((N_E, H), f32)),
  )(x_trace, x_event, wpt, wpe)


def _sc_body(ht4, he4, sf0, sf1, sf2, sf3, sb0, sb1, sb2, sb3,
             dff, dbt, zrow, zcnt, ones_h,
             off0, off1, off2, off3, obt0, obt1, obt2, obt3, ocf, ocb,
             acc, cntf, cntb, sidx, didx, rows0, ones_v, sem0):
  c = lax.axis_index("c")
  s = lax.axis_index("s")
  wid = s * 2 + c
  base = s * RPT

  def do_pass(x_hbm, src_hbm, dst_hbm, cnt_ref):
    pltpu.sync_copy(src_hbm.at[wid], sidx)
    pltpu.sync_copy(dst_hbm.at[wid], didx)
    def chunk(j, carry):
      pltpu.async_copy(x_hbm.at[sidx.at[j]], rows0, sem0).wait()
      pltpu.sync_copy(rows0, acc.at[didx.at[j]], add=True)
      if cnt_ref is not None:
        pltpu.sync_copy(ones_v, cnt_ref.at[didx.at[j]], add=True)
      return carry

    lax.fori_loop(0, NCH, chunk, 0)

  def zero_acc():
    pltpu.sync_copy(zrow.at[pl.ds(base, RPT)], acc.at[pl.ds(base, RPT)])

  zero_acc()
  pltpu.sync_copy(zcnt.at[pl.ds(base, RPT)], cntf.at[pl.ds(base, RPT)])
  pltpu.sync_copy(zcnt.at[pl.ds(base, RPT)], cntb.at[pl.ds(base, RPT)])
  pltpu.sync_copy(ones_h, ones_v)
  plsc.subcore_barrier()

  passes = (
      (ht4, sf0, dff, cntf, off0),
      (ht4, sf1, dff, None, off1),
      (ht4, sf2, dff, None, off2),
      (ht4, sf3, dff, None, off3),
      (he4, sb0, dbt, cntb, obt0),
      (he4, sb1, dbt, None, obt1),
      (he4, sb2, dbt, None, obt2),
      (he4, sb3, dbt, None, obt3),
  )
  for x_hbm, src_hbm, dst_hbm, cnt_ref, out_ref in passes:
    do_pass(x_hbm, src_hbm, dst_hbm, cnt_ref)
    plsc.subcore_barrier()
    pltpu.sync_copy(acc.at[pl.ds(base, RPT)], out_ref.at[c, pl.ds(base, RPT)])
    zero_acc()
    plsc.subcore_barrier()

  pltpu.sync_copy(cntf.at[pl.ds(base, RPT)], ocf.at[c, pl.ds(base, RPT)])
  pltpu.sync_copy(cntb.at[pl.ds(base, RPT)], ocb.at[c, pl.ds(base, RPT)])


def _sc_aggregate(ht4, he4, sff, sbt, dff, dbt, zrow, zcnt, ones_h):
  mesh = plsc.VectorSubcoreMesh(core_axis_name="c", subcore_axis_name="s")
  f32 = jnp.float32
  slab = jax.ShapeDtypeStruct((2, N_PAD, 128), f32)
  cnts = jax.ShapeDtypeStruct((2, N_PAD, 16), f32)
  return pl.kernel(
      _sc_body,
      out_type=(slab, slab, slab, slab, slab, slab, slab, slab, cnts, cnts),
      mesh=mesh,
      scratch_types=[
          pltpu.VMEM_SHARED((N_PAD, 128), f32),
          pltpu.VMEM_SHARED((N_PAD, 16), f32),
          pltpu.VMEM_SHARED((N_PAD, 16), f32),
          pltpu.VMEM((NCH, K), jnp.int32),
          pltpu.VMEM((NCH, K), jnp.int32),
          pltpu.VMEM((K, 128), f32),
          pltpu.VMEM((K, 16), f32),
          pltpu.SemaphoreType.DMA,
      ],
      compiler_params=pltpu.CompilerParams(use_tc_tiling_on_sc=False),
  )(ht4, he4, sff[0], sff[1], sff[2], sff[3], sbt[0], sbt[1], sbt[2], sbt[3],
    dff, dbt, zrow, zcnt, ones_h)


def _tc_body(off0, off1, off2, off3, obt0, obt1, obt2, obt3, cff, cbt,
             ht, tb,
             wlff, wlbt, wrff, wrbt, blff, blbt, wlin, blin, wh, bh,
             head_o, psum, pcnt):
  i = pl.program_id(0)
  f32 = jnp.float32

  @pl.when(i == 0)
  def _():
    psum[...] = jnp.zeros_like(psum)
    pcnt[...] = jnp.zeros_like(pcnt)

  nff = jnp.maximum(cff[0, :, 0] + cff[1, :, 0], 1.0)
  nbt = jnp.maximum(cbt[0, :, 0] + cbt[1, :, 0], 1.0)
  mff = jnp.concatenate(
      [off0[0] + off0[1], off1[0] + off1[1],
       off2[0] + off2[1], off3[0] + off3[1]], axis=1) / nff[:, None]
  mbt = jnp.concatenate(
      [obt0[0] + obt0[1], obt1[0] + obt1[1],
       obt2[0] + obt2[1], obt3[0] + obt3[1]], axis=1) / nbt[:, None]

  dnt = (((1,), (1,)), ((), ()))
  h = ht[...]
  o = (lax.dot_general(mff, wlff[...], dnt, preferred_element_type=f32)
       + blff[...][None, :]
       + lax.dot_general(h, wrff[...], dnt, preferred_element_type=f32)) \
      + (lax.dot_general(mbt, wlbt[...], dnt, preferred_element_type=f32)
         + blbt[...][None, :]
         + lax.dot_general(h, wrbt[...], dnt, preferred_element_type=f32))
  t = jnp.maximum(
      lax.dot_general(o, wlin[...], dnt, preferred_element_type=f32)
      + blin[...][None, :], 0.0)

  ids = tb[0, 0, :]
  p = (ids[:, None] == lax.broadcasted_iota(jnp.int32, (R_BLK, NUM_GRAPHS),
                                            1)).astype(f32)
  psum[...] += lax.dot_general(p, t, (((0,), (0,)), ((), ())),
                               preferred_element_type=f32,
                               precision=lax.Precision.HIGHEST)
  pcnt[...] += jnp.sum(p, axis=0)

  @pl.when(i == N_BLK - 1)
  def _():
    pooled = psum[...] / jnp.maximum(pcnt[...], 1.0)[:, None]
    head_o[...] = (lax.dot_general(pooled, wh[...], dnt,
                                   preferred_element_type=f32)
                   + bh[...][None, :])


def _tc_dense(offs, obts, cff, cbt, h_trace, tb3,
              wlff, wlbt, wrff, wrbt, blff, blbt, wlin, blin, wh, bh):
  f32 = jnp.float32
  full = lambda shp: pl.BlockSpec(shp, lambda i: tuple(0 for _ in shp))
  slab_spec = pl.BlockSpec((2, R_BLK, 128), lambda i: (0, i, 0))
  grid_spec = pltpu.PrefetchScalarGridSpec(
      num_scalar_prefetch=0,
      grid=(N_BLK,),
      in_specs=[
          slab_spec, slab_spec, slab_spec, slab_spec,
          slab_spec, slab_spec, slab_spec, slab_spec,
          pl.BlockSpec((2, R_BLK, 16), lambda i: (0, i, 0)),
          pl.BlockSpec((2, R_BLK, 16), lambda i: (0, i, 0)),
          pl.BlockSpec((R_BLK, H), lambda i: (i, 0)),
          pl.BlockSpec((1, 1, R_BLK), lambda i: (i, 0, 0)),
          full((H, H)), full((H, H)), full((H, H)), full((H, H)),
          full((H,)), full((H,)),
          full((H, H)), full((H,)),
          full((NUM_GRAPHS, H)), full((NUM_GRAPHS,)),
      ],
      out_specs=[
          pl.BlockSpec((NUM_GRAPHS, NUM_GRAPHS), lambda i: (0, 0)),
      ],
      scratch_shapes=[
          pltpu.VMEM((NUM_GRAPHS, H), f32),
          pltpu.VMEM((NUM_GRAPHS,), f32),
      ],
  )
  return pl.pallas_call(
      _tc_body,
      grid_spec=grid_spec,
      out_shape=(jax.ShapeDtypeStruct((NUM_GRAPHS, NUM_GRAPHS), f32),),
  )(offs[0], offs[1], offs[2], offs[3], obts[0], obts[1], obts[2], obts[3],
    cff, cbt, h_trace, tb3,
    wlff, wlbt, wrff, wrbt, blff, blbt, wlin, blin, wh, bh)[0]


def kernel(x_trace, x_event, ei_follows, ei_belongs, ei_contains,
           trace_batch, Wp_trace, Wp_event,
           Wl_ff, bl_ff, Wr_ff, Wl_bt, bl_bt, Wr_bt, Wl_ce, bl_ce, Wr_ce,
           Wlin_trace, blin_trace, Wlin_event, blin_event,
           Wact, bact, Wtime, btime, Wrem, brem):
  i32 = jnp.int32
  f32 = jnp.float32

  h_trace, h_event = _tc_project(x_trace, x_event, Wp_trace, Wp_event)
  ht4 = h_trace.reshape(N_T * 4, 128)
  he4 = h_event.reshape(N_E * 4, 128)

  zpad = jnp.zeros((PAD_E,), i32)
  dpad = DUMP + jnp.arange(PAD_E, dtype=i32) % (N_PAD - DUMP)
  src_ff = jnp.concatenate([ei_follows[0].astype(i32), zpad])
  src_bt = jnp.concatenate([ei_belongs[0].astype(i32), zpad])
  dst_ff = jnp.concatenate([ei_follows[1].astype(i32), dpad]).reshape(
      NW, NCH, K)
  dst_bt = jnp.concatenate([ei_belongs[1].astype(i32), dpad]).reshape(
      NW, NCH, K)
  sff = [(src_ff * 4 + q).reshape(NW, NCH, K) for q in range(4)]
  sbt = [(src_bt * 4 + q).reshape(NW, NCH, K) for q in range(4)]
  zrow = jnp.zeros((N_PAD, 128), f32)
  zcnt = jnp.zeros((N_PAD, 16), f32)
  ones_h = jnp.ones((K, 16), f32)

  outs = _sc_aggregate(ht4, he4, sff, sbt, dst_ff, dst_bt, zrow, zcnt, ones_h)
  offs, obts, cff, cbt = outs[0:4], outs[4:8], outs[8], outs[9]

  tb3 = trace_batch.astype(i32).reshape(N_BLK, 1, R_BLK)
  npad = NUM_GRAPHS - NUM_CLASSES - 2
  wh = jnp.concatenate([Wact, Wtime, Wrem, jnp.zeros((npad, H), f32)], axis=0)
  bh = jnp.concatenate([bact, btime, brem, jnp.zeros((npad,), f32)])
  hout = _tc_dense(
      offs, obts, cff, cbt, h_trace, tb3,
      Wl_ff, Wl_bt, Wr_ff, Wr_bt, bl_ff, bl_bt, Wlin_trace, blin_trace,
      wh, bh)
  return (hout[:, :NUM_CLASSES], hout[:, NUM_CLASSES],
          hout[:, NUM_CLASSES + 1])


# speedup vs baseline: 3.7289x; 3.7289x over previous
import jax
import jax.numpy as jnp
from jax import lax
from jax.experimental import pallas as pl
from jax.experimental.pallas import tpu as pltpu
from jax.experimental.pallas import tpu_sc as plsc

H = 512
N_T = 10000
N_E = 10000
E = 160000
NUM_GRAPHS = 64
NUM_CLASSES = 32

NW = 32
EPW = E // NW
K = 40
NCH = EPW // K
N_PAD = 10240
RPT = N_PAD // 16

R_BLK = 1000
N_BLK = N_T // R_BLK

HI = lax.Precision.HIGHEST


def _sc_body(xt2, he, sf0, sf1, sbt, dff, dbt, zrow, zcnt, ones_h,
             off0, off1, obt, ocf, ocb,
             acc, cntf, cntb, sidx, didx, rows0, ones_v, sem0):
  c = lax.axis_index("c")
  s = lax.axis_index("s")
  wid = s * 2 + c
  base = s * RPT

  def do_pass(x_hbm, src_hbm, dst_hbm, cnt_ref):
    pltpu.sync_copy(src_hbm.at[wid], sidx)
    pltpu.sync_copy(dst_hbm.at[wid], didx)
    def chunk(j, carry):
      pltpu.async_copy(x_hbm.at[sidx.at[j]], rows0, sem0).wait()
      pltpu.sync_copy(rows0, acc.at[didx.at[j]], add=True)
      if cnt_ref is not None:
        pltpu.sync_copy(ones_v, cnt_ref.at[didx.at[j]], add=True)
      return carry

    lax.fori_loop(0, NCH, chunk, 0)

  def zero_acc():
    pltpu.sync_copy(zrow.at[pl.ds(base, RPT)], acc.at[pl.ds(base, RPT)])

  zero_acc()
  pltpu.sync_copy(zcnt.at[pl.ds(base, RPT)], cntf.at[pl.ds(base, RPT)])
  pltpu.sync_copy(zcnt.at[pl.ds(base, RPT)], cntb.at[pl.ds(base, RPT)])
  pltpu.sync_copy(ones_h, ones_v)
  plsc.subcore_barrier()

  passes = (
      (xt2, sf0, dff, cntf, off0),
      (xt2, sf1, dff, None, off1),
      (he, sbt, dbt, cntb, obt),
  )
  for x_hbm, src_hbm, dst_hbm, cnt_ref, out_ref in passes:
    do_pass(x_hbm, src_hbm, dst_hbm, cnt_ref)
    plsc.subcore_barrier()
    pltpu.sync_copy(acc.at[pl.ds(base, RPT)], out_ref.at[c, pl.ds(base, RPT)])
    zero_acc()
    plsc.subcore_barrier()

  pltpu.sync_copy(cntf.at[pl.ds(base, RPT)], ocf.at[c, pl.ds(base, RPT)])
  pltpu.sync_copy(cntb.at[pl.ds(base, RPT)], ocb.at[c, pl.ds(base, RPT)])


def _sc_aggregate(xt2, he, sf0, sf1, sbt, dff, dbt, zrow, zcnt, ones_h):
  mesh = plsc.VectorSubcoreMesh(core_axis_name="c", subcore_axis_name="s")
  f32 = jnp.float32
  slab = jax.ShapeDtypeStruct((2, N_PAD, 128), f32)
  cnts = jax.ShapeDtypeStruct((2, N_PAD, 16), f32)
  return pl.kernel(
      _sc_body,
      out_type=(slab, slab, slab, cnts, cnts),
      mesh=mesh,
      scratch_types=[
          pltpu.VMEM_SHARED((N_PAD, 128), f32),
          pltpu.VMEM_SHARED((N_PAD, 16), f32),
          pltpu.VMEM_SHARED((N_PAD, 16), f32),
          pltpu.VMEM((NCH, K), jnp.int32),
          pltpu.VMEM((NCH, K), jnp.int32),
          pltpu.VMEM((K, 128), f32),
          pltpu.VMEM((K, 16), f32),
          pltpu.SemaphoreType.DMA,
      ],
      compiler_params=pltpu.CompilerParams(use_tc_tiling_on_sc=False),
  )(xt2, he, sf0, sf1, sbt, dff, dbt, zrow, zcnt, ones_h)


def _tc_body(off0, off1, obt, cff, cbt, xt, tb,
             wpt, wpe, wlff, wlbt, wrff, wrbt, blff, blbt, wlin, blin,
             wh, bh,
             head_o, aff, abt, bcm, psum, pcnt):
  i = pl.program_id(0)
  f32 = jnp.float32
  dn = (((1,), (0,)), ((), ()))
  dnt = (((1,), (1,)), ((), ()))

  bf = lambda a: a.astype(jnp.bfloat16).astype(f32)

  @pl.when(i == 0)
  def _():
    aff[...] = lax.dot_general(bf(wlff[...]), bf(wpt[...]), dn,
                               preferred_element_type=f32, precision=HI)
    abt[...] = lax.dot_general(bf(wlbt[...]), bf(wpe[...]), dn,
                               preferred_element_type=f32, precision=HI)
    bcm[...] = lax.dot_general(bf(wrff[...]) + bf(wrbt[...]), bf(wpt[...]), dn,
                               preferred_element_type=f32, precision=HI)
    psum[...] = jnp.zeros_like(psum)
    pcnt[...] = jnp.zeros_like(pcnt)

  nff = jnp.maximum(cff[0, :, 0] + cff[1, :, 0], 1.0)
  nbt = jnp.maximum(cbt[0, :, 0] + cbt[1, :, 0], 1.0)
  mff = jnp.concatenate(
      [off0[0] + off0[1], off1[0] + off1[1]], axis=1) / nff[:, None]
  mbt = (obt[0] + obt[1]) / nbt[:, None]

  o = (lax.dot_general(mff, aff[...], dnt, preferred_element_type=f32,
                       precision=HI)
       + lax.dot_general(mbt, abt[...], dnt, preferred_element_type=f32,
                         precision=HI)
       + lax.dot_general(xt[...], bcm[...], dnt, preferred_element_type=f32,
                         precision=HI)
       + (blff[...] + blbt[...])[None, :])
  t = jnp.maximum(
      lax.dot_general(o, bf(wlin[...]), dnt, preferred_element_type=f32,
                      precision=HI)
      + blin[...][None, :], 0.0)

  ids = tb[0, 0, :]
  p = (ids[:, None] == lax.broadcasted_iota(jnp.int32, (R_BLK, NUM_GRAPHS),
                                            1)).astype(f32)
  psum[...] += lax.dot_general(p, t, (((0,), (0,)), ((), ())),
                               preferred_element_type=f32, precision=HI)
  pcnt[...] += jnp.sum(p, axis=0)

  @pl.when(i == N_BLK - 1)
  def _():
    pooled = psum[...] / jnp.maximum(pcnt[...], 1.0)[:, None]
    head_o[...] = (lax.dot_general(bf(pooled), bf(wh[...]), dnt,
                                   preferred_element_type=f32, precision=HI)
                   + bh[...][None, :])


def _tc_dense(off0, off1, obt, cff, cbt, x_trace, tb3,
              wpt, wpe, wlff, wlbt, wrff, wrbt, blff, blbt, wlin, blin,
              wh, bh):
  f32 = jnp.float32
  full = lambda shp: pl.BlockSpec(shp, lambda i: tuple(0 for _ in shp))
  slab_spec = pl.BlockSpec((2, R_BLK, 128), lambda i: (0, i, 0))
  grid_spec = pltpu.PrefetchScalarGridSpec(
      num_scalar_prefetch=0,
      grid=(N_BLK,),
      in_specs=[
          slab_spec, slab_spec, slab_spec,
          pl.BlockSpec((2, R_BLK, 16), lambda i: (0, i, 0)),
          pl.BlockSpec((2, R_BLK, 16), lambda i: (0, i, 0)),
          pl.BlockSpec((R_BLK, 256), lambda i: (i, 0)),
          pl.BlockSpec((1, 1, R_BLK), lambda i: (i, 0, 0)),
          full((H, 256)), full((H, 128)),
          full((H, H)), full((H, H)), full((H, H)), full((H, H)),
          full((H,)), full((H,)),
          full((H, H)), full((H,)),
          full((NUM_GRAPHS, H)), full((NUM_GRAPHS,)),
      ],
      out_specs=[
          pl.BlockSpec((NUM_GRAPHS, NUM_GRAPHS), lambda i: (0, 0)),
      ],
      scratch_shapes=[
          pltpu.VMEM((H, 256), f32),
          pltpu.VMEM((H, 128), f32),
          pltpu.VMEM((H, 256), f32),
          pltpu.VMEM((NUM_GRAPHS, H), f32),
          pltpu.VMEM((NUM_GRAPHS,), f32),
      ],
  )
  return pl.pallas_call(
      _tc_body,
      grid_spec=grid_spec,
      out_shape=(jax.ShapeDtypeStruct((NUM_GRAPHS, NUM_GRAPHS), f32),),
  )(off0, off1, obt, cff, cbt, x_trace, tb3,
    wpt, wpe, wlff, wlbt, wrff, wrbt, blff, blbt, wlin, blin, wh, bh)[0]


def kernel(x_trace, x_event, ei_follows, ei_belongs, ei_contains,
           trace_batch, Wp_trace, Wp_event,
           Wl_ff, bl_ff, Wr_ff, Wl_bt, bl_bt, Wr_bt, Wl_ce, bl_ce, Wr_ce,
           Wlin_trace, blin_trace, Wlin_event, blin_event,
           Wact, bact, Wtime, btime, Wrem, brem):
  i32 = jnp.int32
  f32 = jnp.float32

  xt2 = x_trace.reshape(N_T * 2, 128)
  src_ff = ei_follows[0].astype(i32)
  sf0 = (src_ff * 2).reshape(NW, NCH, K)
  sf1 = (src_ff * 2 + 1).reshape(NW, NCH, K)
  sbt = ei_belongs[0].astype(i32).reshape(NW, NCH, K)
  dff = ei_follows[1].astype(i32).reshape(NW, NCH, K)
  dbt = ei_belongs[1].astype(i32).reshape(NW, NCH, K)
  zrow = jnp.zeros((N_PAD, 128), f32)
  zcnt = jnp.zeros((N_PAD, 16), f32)
  ones_h = jnp.ones((K, 16), f32)

  off0, off1, obt, cff, cbt = _sc_aggregate(
      xt2, x_event, sf0, sf1, sbt, dff, dbt, zrow, zcnt, ones_h)

  tb3 = trace_batch.astype(i32).reshape(N_BLK, 1, R_BLK)
  npad = NUM_GRAPHS - NUM_CLASSES - 2
  wh = jnp.concatenate([Wact, Wtime, Wrem, jnp.zeros((npad, H), f32)], axis=0)
  bh = jnp.concatenate([bact, btime, brem, jnp.zeros((npad,), f32)])
  hout = _tc_dense(
      off0, off1, obt, cff, cbt, x_trace, tb3,
      Wp_trace, Wp_event, Wl_ff, Wl_bt, Wr_ff, Wr_bt, bl_ff, bl_bt,
      Wlin_trace, blin_trace, wh, bh)
  return (hout[:, :NUM_CLASSES], hout[:, NUM_CLASSES],
          hout[:, NUM_CLASSES + 1])
